# baseline (speedup 1.0000x reference)
import functools

import jax
import jax.numpy as jnp
from jax import lax
from jax.experimental import pallas as pl
from jax.experimental.pallas import tpu as pltpu
from jax.experimental.pallas import tpu_sc as plsc

NC = 2
NS = 16
NW = NC * NS
LANES = 16
EB = 128
WIN = 8


def _mesh():
    return plsc.VectorSubcoreMesh(core_axis_name="c", subcore_axis_name="s")


def _row_split(n):
    per = (n // NS) & ~7
    return per, n - per * (NS - 1)


def _deg_pass(ei, n):
    e = ei.shape[1]
    nbt = e // EB
    nbf = nbt // NW
    rem = nbt - nbf * NW

    assert nbf % 4 == 2 and nbf >= 6

    @functools.partial(
        pl.kernel,
        out_type=jax.ShapeDtypeStruct((NC, 1, n), jnp.float32),
        mesh=_mesh(),
        scratch_types=[
            pltpu.VMEM_SHARED((n,), jnp.float32),
            pltpu.VMEM((n,), jnp.float32),
            pltpu.VMEM((EB,), jnp.float32),
            pltpu.VMEM((2, EB), jnp.int32),
            pltpu.VMEM((2, EB), jnp.int32),
            pltpu.VMEM((2, EB), jnp.int32),
            pltpu.VMEM((2, EB), jnp.int32),
            [pltpu.SemaphoreType.DMA] * 4,
            [pltpu.SemaphoreType.DMA] * 4,
        ],
    )
    def deg_kernel(ei_hbm, deg_out, deg_sh, zbuf, ones_v, db0, db1, db2, db3,
                   lsem, ssem):
        c = lax.axis_index("c")
        s = lax.axis_index("s")
        w = s * NC + c
        nb = nbf + jnp.where(w < rem, 1, 0)
        start = w * nbf + jnp.minimum(w, rem)
        db = (db0, db1, db2, db3)

        def _ld(i, b):
            pltpu.async_copy(
                ei_hbm.at[pl.ds(0, 2), pl.ds((start + i) * EB, EB)],
                db[b], lsem[b])

        def _ld_wait(b):
            pltpu.make_async_copy(ei_hbm.at[pl.ds(0, 2), pl.ds(0, EB)],
                                  db[b], lsem[b]).wait()

        def _sc(i, b):
            pltpu.async_copy(ones_v, deg_sh.at[db[b].at[1]], ssem[b],
                             add=True)

        def _sc_wait(b):
            pltpu.make_async_copy(ones_v, deg_sh.at[db[0].at[1]],
                                  ssem[b]).wait()

        _ld(0, 0)
        _ld(1, 1)

        def fill1(i, carry):
            ones_v[pl.ds(i * LANES, LANES)] = jnp.ones((LANES,), jnp.float32)
            return carry

        lax.fori_loop(0, EB // LANES, fill1, 0)

        @pl.when(s == 0)
        def _():
            def fill(i, carry):
                zbuf[pl.ds(i * LANES, LANES)] = jnp.zeros((LANES,),
                                                          jnp.float32)
                return carry

            lax.fori_loop(0, n // LANES, fill, 0)
            pltpu.sync_copy(zbuf, deg_sh)

        plsc.subcore_barrier()

        def step(i, b):
            _ld_wait(b)
            _sc(i, b)

            @pl.when(i + 2 < nb)
            def _(i=i, b=b):
                @pl.when(i >= 2)
                def _():
                    _sc_wait((b + 2) % 4)

                _ld(i + 2, (b + 2) % 4)

        def body(j, carry):
            for k in range(4):
                step(4 * j + k, k)
            return carry

        lax.fori_loop(0, nbf // 4, body, 0)
        step(nbf - 2, 0)
        step(nbf - 1, 1)

        @pl.when(w < rem)
        def _():
            _ld_wait(2)
            _sc(nbf, 2)

        for k in range(4):
            _sc_wait(k)

        plsc.subcore_barrier()

        @pl.when(s == 0)
        def _():
            pltpu.sync_copy(deg_sh, deg_out.at[c, 0])

    return deg_kernel(ei)


def _gs_pass(g, ei):
    n, d = g.shape
    e = ei.shape[1]
    nbt = e // EB
    nbf = nbt // NW
    rem = nbt - nbf * NW
    per, last = _row_split(n)
    assert nbf % 4 == 2 and nbf >= 6

    @functools.partial(
        pl.kernel,
        out_type=jax.ShapeDtypeStruct((NC, n, d), jnp.float32),
        mesh=_mesh(),
        scratch_types=[
            pltpu.VMEM_SHARED((n, d), jnp.float32),
            pltpu.VMEM((2, EB), jnp.int32),
            pltpu.VMEM((2, EB), jnp.int32),
            pltpu.VMEM((2, EB), jnp.int32),
            pltpu.VMEM((2, EB), jnp.int32),
            pltpu.VMEM((EB, d), jnp.float32),
            pltpu.VMEM((EB, d), jnp.float32),
            pltpu.SemaphoreType.DMA,
            pltpu.SemaphoreType.DMA,
            pltpu.SemaphoreType.DMA,
            pltpu.SemaphoreType.DMA,
            pltpu.SemaphoreType.DMA,
            pltpu.SemaphoreType.DMA,
        ],
    )
    def gs_kernel(g_hbm, ei_hbm, acc_out, acc_sh, ib0, ib1, ib2, ib3,
                  rows0, rows1, semg0, semg1, il0, il1, il2, il3):
        c = lax.axis_index("c")
        s = lax.axis_index("s")
        w = s * NC + c
        nb = nbf + jnp.where(w < rem, 1, 0)
        start = w * nbf + jnp.minimum(w, rem)
        base = pl.multiple_of(s * per, 8)

        @pl.when(s < NS - 1)
        def _():
            pltpu.sync_copy(g_hbm.at[pl.ds(base, per)],
                            acc_sh.at[pl.ds(base, per)])

        @pl.when(s == NS - 1)
        def _():
            pltpu.sync_copy(g_hbm.at[pl.ds(base, last)],
                            acc_sh.at[pl.ds(base, last)])

        plsc.subcore_barrier()

        rows = (rows0, rows1)
        ib = (ib0, ib1, ib2, ib3)
        semg = (semg0, semg1)
        il = (il0, il1, il2, il3)

        def _iload(i, b4):
            pltpu.async_copy(
                ei_hbm.at[pl.ds(0, 2), pl.ds((start + i) * EB, EB)],
                ib[b4], il[b4])

        def _iload_wait(b4):
            pltpu.make_async_copy(ei_hbm.at[pl.ds(0, 2), pl.ds(0, EB)],
                                  ib[b4], il[b4]).wait()

        def _gather(i, b2, b4):
            pltpu.async_copy(g_hbm.at[ib[b4].at[0]], rows[b2], semg[b2])

        def _gather_wait(b2, b4):
            pltpu.make_async_copy(g_hbm.at[ib[b4].at[0]], rows[b2],
                                  semg[b2]).wait()

        def _batch(i, b2, b4):
            @pl.when(i + 2 < nb)
            def _():
                _iload(i + 2, (b4 + 2) % 4)

            _gather_wait(b2, b4)
            pltpu.sync_copy(rows[b2], acc_sh.at[ib[b4].at[1]], add=True)

            @pl.when(i + 2 < nb)
            def _():
                _iload_wait((b4 + 2) % 4)
                _gather(i + 2, b2, (b4 + 2) % 4)

        _iload(0, 0)
        _iload(1, 1)
        _iload_wait(0)
        _gather(0, 0, 0)
        _iload_wait(1)
        _gather(1, 1, 1)

        def body(j, carry):
            for k in range(4):
                i = 4 * j + k
                _batch(i, k % 2, k)
            return carry

        lax.fori_loop(0, nbf // 4, body, 0)

        _batch(nbf - 2, 0, 0)
        _batch(nbf - 1, 1, 1)

        @pl.when(w < rem)
        def _():
            _gather_wait(0, 2)
            pltpu.sync_copy(rows[0], acc_sh.at[ib[2].at[1]], add=True)

        plsc.subcore_barrier()

        @pl.when(s < NS - 1)
        def _():
            pltpu.sync_copy(acc_sh.at[pl.ds(base, per)],
                            acc_out.at[c, pl.ds(base, per)])

        @pl.when(s == NS - 1)
        def _():
            pltpu.sync_copy(acc_sh.at[pl.ds(base, last)],
                            acc_out.at[c, pl.ds(base, last)])

    return gs_kernel(g, ei)


def _tc_b1(x, w):
    n, d_in = x.shape
    d_out = w.shape[1]
    blk = 2000

    def body(x_ref, w_ref, out_ref):
        xb = x_ref[...]
        nrm = jnp.sqrt(jnp.sum(xb * xb, axis=1, keepdims=True))
        xn = xb / (nrm + 1e-8)
        out_ref[...] = jnp.dot(xn, w_ref[...],
                               preferred_element_type=jnp.float32)

    return pl.pallas_call(
        body,
        grid=(n // blk,),
        in_specs=[
            pl.BlockSpec((blk, d_in), lambda i: (i, 0)),
            pl.BlockSpec((d_in, d_out), lambda i: (0, 0)),
        ],
        out_specs=pl.BlockSpec((blk, d_out), lambda i: (i, 0)),
        out_shape=jax.ShapeDtypeStruct((n, d_out), jnp.float32),
    )(x, w)


def _tc_b2(h, dsum):
    n, d = h.shape
    blk = 2000

    def body(h_ref, d_ref, out_ref):
        dinv = 1.0 / jnp.sqrt(d_ref[...] + 1.0)
        out_ref[...] = h_ref[...] * dinv

    return pl.pallas_call(
        body,
        grid=(n // blk,),
        in_specs=[
            pl.BlockSpec((blk, d), lambda i: (i, 0)),
            pl.BlockSpec((blk, 1), lambda i: (i, 0)),
        ],
        out_specs=pl.BlockSpec((blk, d), lambda i: (i, 0)),
        out_shape=jax.ShapeDtypeStruct((n, d), jnp.float32),
    )(h, dsum)


def _tc_d(acc, g, dsum, b2):
    n, d = g.shape
    blk = 2000

    def body(acc_ref, g_ref, d_ref, b_ref, out_ref):
        dinv = 1.0 / jnp.sqrt(d_ref[...] + 1.0)
        tot = acc_ref[0] + acc_ref[1] - g_ref[...]
        out_ref[...] = tot * dinv + b_ref[...]

    return pl.pallas_call(
        body,
        grid=(n // blk,),
        in_specs=[
            pl.BlockSpec((NC, blk, d), lambda i: (0, i, 0)),
            pl.BlockSpec((blk, d), lambda i: (i, 0)),
            pl.BlockSpec((blk, 1), lambda i: (i, 0)),
            pl.BlockSpec((1, d), lambda i: (0, 0)),
        ],
        out_specs=pl.BlockSpec((blk, d), lambda i: (i, 0)),
        out_shape=jax.ShapeDtypeStruct((n, d), jnp.float32),
    )(acc, g, dsum, b2)


def kernel(x, edge_index, W, b):
    n, _ = x.shape
    e = edge_index.shape[1]
    assert e % EB == 0 and n % LANES == 0
    h = _tc_b1(x, W)
    deg_p = _deg_pass(edge_index, n)
    dsum = (deg_p[0, 0] + deg_p[1, 0]).reshape(n, 1)
    g = _tc_b2(h, dsum)
    acc = _gs_pass(g, edge_index)
    return _tc_d(acc, g, dsum, b.reshape(1, -1))


# speedup vs baseline: 1.0727x; 1.0727x over previous
import functools

import jax
import jax.numpy as jnp
from jax import lax
from jax.experimental import pallas as pl
from jax.experimental.pallas import tpu as pltpu
from jax.experimental.pallas import tpu_sc as plsc

NC = 2
NS = 16
NW = NC * NS
LANES = 16
EB = 128
WIN = 8


def _mesh():
    return plsc.VectorSubcoreMesh(core_axis_name="c", subcore_axis_name="s")


def _row_split(n):
    per = (n // NS) & ~7
    return per, n - per * (NS - 1)


def _deg_pass(ei, n):
    e = ei.shape[1]
    nbt = e // EB
    nbf = nbt // NW
    rem = nbt - nbf * NW

    @functools.partial(
        pl.kernel,
        out_type=jax.ShapeDtypeStruct((NC, 1, n), jnp.float32),
        mesh=_mesh(),
        scratch_types=[
            pltpu.VMEM_SHARED((n,), jnp.float32),
            pltpu.VMEM((n,), jnp.float32),
            pltpu.VMEM((EB,), jnp.float32),
            pltpu.VMEM((nbf + 1, 2, EB), jnp.int32),
            pltpu.SemaphoreType.DMA,
            pltpu.SemaphoreType.DMA,
        ],
    )
    def deg_kernel(ei_hbm, deg_out, deg_sh, zbuf, ones_v, didx, seml, sems):
        c = lax.axis_index("c")
        s = lax.axis_index("s")
        w = s * NC + c
        nb = nbf + jnp.where(w < rem, 1, 0)
        start = w * nbf + jnp.minimum(w, rem)

        def fill1(i, carry):
            ones_v[pl.ds(i * LANES, LANES)] = jnp.ones((LANES,), jnp.float32)
            return carry

        lax.fori_loop(0, EB // LANES, fill1, 0)

        @pl.when(s == 0)
        def _():
            def fill(i, carry):
                zbuf[pl.ds(i * LANES, LANES)] = jnp.zeros((LANES,),
                                                          jnp.float32)
                return carry

            lax.fori_loop(0, n // LANES, fill, 0)
            pltpu.sync_copy(zbuf, deg_sh)

        def stage(i, carry):
            pltpu.async_copy(
                ei_hbm.at[pl.ds(0, 2), pl.ds((start + i) * EB, EB)],
                didx.at[i], seml)

            @pl.when(i >= WIN)
            def _():
                pltpu.make_async_copy(ei_hbm.at[pl.ds(0, 2), pl.ds(0, EB)],
                                      didx.at[0], seml).wait()

            return carry

        lax.fori_loop(0, nb, stage, 0)
        for _k in range(WIN):
            pltpu.make_async_copy(ei_hbm.at[pl.ds(0, 2), pl.ds(0, EB)],
                                  didx.at[0], seml).wait()

        plsc.subcore_barrier()

        def body(i, carry):
            pltpu.async_copy(ones_v, deg_sh.at[didx.at[i, 1]], sems, add=True)

            @pl.when(i >= WIN)
            def _():
                pltpu.make_async_copy(ones_v, deg_sh.at[didx.at[0, 1]],
                                      sems).wait()

            return carry

        lax.fori_loop(0, nb, body, 0)
        for _k in range(WIN):
            pltpu.make_async_copy(ones_v, deg_sh.at[didx.at[0, 1]],
                                  sems).wait()

        plsc.subcore_barrier()

        @pl.when(s == 0)
        def _():
            pltpu.sync_copy(deg_sh, deg_out.at[c, 0])

    return deg_kernel(ei)


def _gs_pass(g, ei):
    n, d = g.shape
    e = ei.shape[1]
    nbt = e // EB
    nbf = nbt // NW
    rem = nbt - nbf * NW
    per, last = _row_split(n)
    assert nbf % 4 == 2 and nbf >= 6

    @functools.partial(
        pl.kernel,
        out_type=jax.ShapeDtypeStruct((NC, n, d), jnp.float32),
        mesh=_mesh(),
        scratch_types=[
            pltpu.VMEM_SHARED((n, d), jnp.float32),
            pltpu.VMEM((2, EB), jnp.int32),
            pltpu.VMEM((2, EB), jnp.int32),
            pltpu.VMEM((2, EB), jnp.int32),
            pltpu.VMEM((2, EB), jnp.int32),
            pltpu.VMEM((EB, d), jnp.float32),
            pltpu.VMEM((EB, d), jnp.float32),
            pltpu.SemaphoreType.DMA,
            pltpu.SemaphoreType.DMA,
            pltpu.SemaphoreType.DMA,
            pltpu.SemaphoreType.DMA,
            pltpu.SemaphoreType.DMA,
            pltpu.SemaphoreType.DMA,
        ],
    )
    def gs_kernel(g_hbm, ei_hbm, acc_out, acc_sh, ib0, ib1, ib2, ib3,
                  rows0, rows1, semg0, semg1, il0, il1, il2, il3):
        c = lax.axis_index("c")
        s = lax.axis_index("s")
        w = s * NC + c
        nb = nbf + jnp.where(w < rem, 1, 0)
        start = w * nbf + jnp.minimum(w, rem)
        base = pl.multiple_of(s * per, 8)

        @pl.when(s < NS - 1)
        def _():
            pltpu.sync_copy(g_hbm.at[pl.ds(base, per)],
                            acc_sh.at[pl.ds(base, per)])

        @pl.when(s == NS - 1)
        def _():
            pltpu.sync_copy(g_hbm.at[pl.ds(base, last)],
                            acc_sh.at[pl.ds(base, last)])

        plsc.subcore_barrier()

        rows = (rows0, rows1)
        ib = (ib0, ib1, ib2, ib3)
        semg = (semg0, semg1)
        il = (il0, il1, il2, il3)

        def _iload(i, b4):
            pltpu.async_copy(
                ei_hbm.at[pl.ds(0, 2), pl.ds((start + i) * EB, EB)],
                ib[b4], il[b4])

        def _iload_wait(b4):
            pltpu.make_async_copy(ei_hbm.at[pl.ds(0, 2), pl.ds(0, EB)],
                                  ib[b4], il[b4]).wait()

        def _gather(i, b2, b4):
            pltpu.async_copy(g_hbm.at[ib[b4].at[0]], rows[b2], semg[b2])

        def _gather_wait(b2, b4):
            pltpu.make_async_copy(g_hbm.at[ib[b4].at[0]], rows[b2],
                                  semg[b2]).wait()

        def _batch(i, b2, b4):
            @pl.when(i + 2 < nb)
            def _():
                _iload(i + 2, (b4 + 2) % 4)

            _gather_wait(b2, b4)
            pltpu.sync_copy(rows[b2], acc_sh.at[ib[b4].at[1]], add=True)

            @pl.when(i + 2 < nb)
            def _():
                _iload_wait((b4 + 2) % 4)
                _gather(i + 2, b2, (b4 + 2) % 4)

        _iload(0, 0)
        _iload(1, 1)
        _iload_wait(0)
        _gather(0, 0, 0)
        _iload_wait(1)
        _gather(1, 1, 1)

        def body(j, carry):
            for k in range(4):
                i = 4 * j + k
                _batch(i, k % 2, k)
            return carry

        lax.fori_loop(0, nbf // 4, body, 0)

        _batch(nbf - 2, 0, 0)
        _batch(nbf - 1, 1, 1)

        @pl.when(w < rem)
        def _():
            _gather_wait(0, 2)
            pltpu.sync_copy(rows[0], acc_sh.at[ib[2].at[1]], add=True)

        plsc.subcore_barrier()

        @pl.when(s < NS - 1)
        def _():
            pltpu.sync_copy(acc_sh.at[pl.ds(base, per)],
                            acc_out.at[c, pl.ds(base, per)])

        @pl.when(s == NS - 1)
        def _():
            pltpu.sync_copy(acc_sh.at[pl.ds(base, last)],
                            acc_out.at[c, pl.ds(base, last)])

    return gs_kernel(g, ei)


def _tc_b1(x, w):
    n, d_in = x.shape
    d_out = w.shape[1]
    blk = 2000

    def body(x_ref, w_ref, out_ref):
        xb = x_ref[...]
        nrm = jnp.sqrt(jnp.sum(xb * xb, axis=1, keepdims=True))
        xn = xb / (nrm + 1e-8)
        out_ref[...] = jnp.dot(xn, w_ref[...],
                               preferred_element_type=jnp.float32)

    return pl.pallas_call(
        body,
        grid=(n // blk,),
        in_specs=[
            pl.BlockSpec((blk, d_in), lambda i: (i, 0)),
            pl.BlockSpec((d_in, d_out), lambda i: (0, 0)),
        ],
        out_specs=pl.BlockSpec((blk, d_out), lambda i: (i, 0)),
        out_shape=jax.ShapeDtypeStruct((n, d_out), jnp.float32),
    )(x, w)


def _tc_b2(h, dsum):
    n, d = h.shape
    blk = 2000

    def body(h_ref, d_ref, out_ref):
        dinv = 1.0 / jnp.sqrt(d_ref[...] + 1.0)
        out_ref[...] = h_ref[...] * dinv

    return pl.pallas_call(
        body,
        grid=(n // blk,),
        in_specs=[
            pl.BlockSpec((blk, d), lambda i: (i, 0)),
            pl.BlockSpec((blk, 1), lambda i: (i, 0)),
        ],
        out_specs=pl.BlockSpec((blk, d), lambda i: (i, 0)),
        out_shape=jax.ShapeDtypeStruct((n, d), jnp.float32),
    )(h, dsum)


def _tc_d(acc, g, dsum, b2):
    n, d = g.shape
    blk = 2000

    def body(acc_ref, g_ref, d_ref, b_ref, out_ref):
        dinv = 1.0 / jnp.sqrt(d_ref[...] + 1.0)
        tot = acc_ref[0] + acc_ref[1] - g_ref[...]
        out_ref[...] = tot * dinv + b_ref[...]

    return pl.pallas_call(
        body,
        grid=(n // blk,),
        in_specs=[
            pl.BlockSpec((NC, blk, d), lambda i: (0, i, 0)),
            pl.BlockSpec((blk, d), lambda i: (i, 0)),
            pl.BlockSpec((blk, 1), lambda i: (i, 0)),
            pl.BlockSpec((1, d), lambda i: (0, 0)),
        ],
        out_specs=pl.BlockSpec((blk, d), lambda i: (i, 0)),
        out_shape=jax.ShapeDtypeStruct((n, d), jnp.float32),
    )(acc, g, dsum, b2)


def kernel(x, edge_index, W, b):
    n, _ = x.shape
    e = edge_index.shape[1]
    assert e % EB == 0 and n % LANES == 0
    h = _tc_b1(x, W)
    deg_p = _deg_pass(edge_index, n)
    dsum = (deg_p[0, 0] + deg_p[1, 0]).reshape(n, 1)
    g = _tc_b2(h, dsum)
    acc = _gs_pass(g, edge_index)
    return _tc_d(acc, g, dsum, b.reshape(1, -1))


# speedup vs baseline: 1.0732x; 1.0005x over previous
import functools

import jax
import jax.numpy as jnp
from jax import lax
from jax.experimental import pallas as pl
from jax.experimental.pallas import tpu as pltpu
from jax.experimental.pallas import tpu_sc as plsc

NC = 2
NS = 16
NW = NC * NS
LANES = 16
EB = 128
WIN = 8


def _mesh():
    return plsc.VectorSubcoreMesh(core_axis_name="c", subcore_axis_name="s")


def _row_split(n):
    per = (n // NS) & ~7
    return per, n - per * (NS - 1)


def _deg_pass(ei, n):
    e = ei.shape[1]
    nbt = e // EB
    nbf = nbt // NW
    rem = nbt - nbf * NW

    @functools.partial(
        pl.kernel,
        out_type=jax.ShapeDtypeStruct((NC, 1, n), jnp.float32),
        mesh=_mesh(),
        scratch_types=[
            pltpu.VMEM_SHARED((n,), jnp.float32),
            pltpu.VMEM((n,), jnp.float32),
            pltpu.VMEM((EB,), jnp.float32),
            pltpu.VMEM((nbf + 1, 2, EB), jnp.int32),
            pltpu.SemaphoreType.DMA,
            pltpu.SemaphoreType.DMA,
        ],
    )
    def deg_kernel(ei_hbm, deg_out, deg_sh, zbuf, ones_v, didx, seml, sems):
        c = lax.axis_index("c")
        s = lax.axis_index("s")
        w = s * NC + c
        nb = nbf + jnp.where(w < rem, 1, 0)
        start = w * nbf + jnp.minimum(w, rem)

        def fill1(i, carry):
            ones_v[pl.ds(i * LANES, LANES)] = jnp.ones((LANES,), jnp.float32)
            return carry

        lax.fori_loop(0, EB // LANES, fill1, 0)

        @pl.when(s == 0)
        def _():
            def fill(i, carry):
                zbuf[pl.ds(i * LANES, LANES)] = jnp.zeros((LANES,),
                                                          jnp.float32)
                return carry

            lax.fori_loop(0, n // LANES, fill, 0)
            pltpu.sync_copy(zbuf, deg_sh)

        def stage(i, carry):
            pltpu.async_copy(
                ei_hbm.at[pl.ds(0, 2), pl.ds((start + i) * EB, EB)],
                didx.at[i], seml)

            @pl.when(i >= WIN)
            def _():
                pltpu.make_async_copy(ei_hbm.at[pl.ds(0, 2), pl.ds(0, EB)],
                                      didx.at[0], seml).wait()

            return carry

        lax.fori_loop(0, nb, stage, 0)
        for _k in range(WIN):
            pltpu.make_async_copy(ei_hbm.at[pl.ds(0, 2), pl.ds(0, EB)],
                                  didx.at[0], seml).wait()

        plsc.subcore_barrier()

        def body(i, carry):
            pltpu.async_copy(ones_v, deg_sh.at[didx.at[i, 1]], sems, add=True)

            @pl.when(i >= WIN)
            def _():
                pltpu.make_async_copy(ones_v, deg_sh.at[didx.at[0, 1]],
                                      sems).wait()

            return carry

        lax.fori_loop(0, nb, body, 0)
        for _k in range(WIN):
            pltpu.make_async_copy(ones_v, deg_sh.at[didx.at[0, 1]],
                                  sems).wait()

        plsc.subcore_barrier()

        @pl.when(s == 0)
        def _():
            pltpu.sync_copy(deg_sh, deg_out.at[c, 0])

    return deg_kernel(ei)


def _gs_pass(g, ei):
    n, d = g.shape
    e = ei.shape[1]
    nbt = e // EB
    nbf = nbt // NW
    rem = nbt - nbf * NW
    per, last = _row_split(n)
    assert nbf % 4 == 2 and nbf >= 6

    @functools.partial(
        pl.kernel,
        out_type=jax.ShapeDtypeStruct((NC, n, d), jnp.float32),
        mesh=_mesh(),
        scratch_types=[
            pltpu.VMEM_SHARED((n, d), jnp.float32),
            pltpu.VMEM((2, EB), jnp.int32),
            pltpu.VMEM((2, EB), jnp.int32),
            pltpu.VMEM((2, EB), jnp.int32),
            pltpu.VMEM((2, EB), jnp.int32),
            pltpu.VMEM((EB, d), jnp.float32),
            pltpu.VMEM((EB, d), jnp.float32),
            pltpu.SemaphoreType.DMA,
            pltpu.SemaphoreType.DMA,
            pltpu.SemaphoreType.DMA,
            pltpu.SemaphoreType.DMA,
            pltpu.SemaphoreType.DMA,
            pltpu.SemaphoreType.DMA,
            pltpu.SemaphoreType.DMA,
            pltpu.SemaphoreType.DMA,
        ],
    )
    def gs_kernel(g_hbm, ei_hbm, acc_out, acc_sh, ib0, ib1, ib2, ib3,
                  rows0, rows1, semg0, semg1, il0, il1, il2, il3, ss0, ss1):
        c = lax.axis_index("c")
        s = lax.axis_index("s")
        w = s * NC + c
        nb = nbf + jnp.where(w < rem, 1, 0)
        start = w * nbf + jnp.minimum(w, rem)
        base = pl.multiple_of(s * per, 8)

        @pl.when(s < NS - 1)
        def _():
            pltpu.sync_copy(g_hbm.at[pl.ds(base, per)],
                            acc_sh.at[pl.ds(base, per)])

        @pl.when(s == NS - 1)
        def _():
            pltpu.sync_copy(g_hbm.at[pl.ds(base, last)],
                            acc_sh.at[pl.ds(base, last)])

        plsc.subcore_barrier()

        rows = (rows0, rows1)
        ib = (ib0, ib1, ib2, ib3)
        semg = (semg0, semg1)
        il = (il0, il1, il2, il3)
        sems = (ss0, ss1)

        def _scatter(b2, b4):
            pltpu.async_copy(rows[b2], acc_sh.at[ib[b4].at[1]], sems[b2],
                             add=True)
            pltpu.make_async_copy(rows[b2], acc_sh.at[ib[0].at[1]],
                                  sems[b2]).wait()

        def _iload(i, b4):
            pltpu.async_copy(
                ei_hbm.at[pl.ds(0, 2), pl.ds((start + i) * EB, EB)],
                ib[b4], il[b4])

        def _iload_wait(b4):
            pltpu.make_async_copy(ei_hbm.at[pl.ds(0, 2), pl.ds(0, EB)],
                                  ib[b4], il[b4]).wait()

        def _gather(i, b2, b4):
            pltpu.async_copy(g_hbm.at[ib[b4].at[0]], rows[b2], semg[b2])

        def _gather_wait(b2, b4):
            pltpu.make_async_copy(g_hbm.at[ib[b4].at[0]], rows[b2],
                                  semg[b2]).wait()

        def _batch(i, b2, b4):
            @pl.when(i + 2 < nb)
            def _():
                _iload(i + 2, (b4 + 2) % 4)

            _gather_wait(b2, b4)
            _scatter(b2, b4)

            @pl.when(i + 2 < nb)
            def _():
                _iload_wait((b4 + 2) % 4)
                _gather(i + 2, b2, (b4 + 2) % 4)

        _iload(0, 0)
        _iload(1, 1)
        _iload_wait(0)
        _gather(0, 0, 0)
        _iload_wait(1)
        _gather(1, 1, 1)

        def body(j, carry):
            for k in range(4):
                i = 4 * j + k
                _batch(i, k % 2, k)
            return carry

        lax.fori_loop(0, nbf // 4, body, 0)

        _batch(nbf - 2, 0, 0)
        _batch(nbf - 1, 1, 1)

        @pl.when(w < rem)
        def _():
            _gather_wait(0, 2)
            _scatter(0, 2)

        plsc.subcore_barrier()

        @pl.when(s < NS - 1)
        def _():
            pltpu.sync_copy(acc_sh.at[pl.ds(base, per)],
                            acc_out.at[c, pl.ds(base, per)])

        @pl.when(s == NS - 1)
        def _():
            pltpu.sync_copy(acc_sh.at[pl.ds(base, last)],
                            acc_out.at[c, pl.ds(base, last)])

    return gs_kernel(g, ei)


def _tc_b1(x, w):
    n, d_in = x.shape
    d_out = w.shape[1]
    blk = 2000

    def body(x_ref, w_ref, out_ref):
        xb = x_ref[...]
        nrm = jnp.sqrt(jnp.sum(xb * xb, axis=1, keepdims=True))
        xn = xb / (nrm + 1e-8)
        out_ref[...] = jnp.dot(xn, w_ref[...],
                               preferred_element_type=jnp.float32)

    return pl.pallas_call(
        body,
        grid=(n // blk,),
        in_specs=[
            pl.BlockSpec((blk, d_in), lambda i: (i, 0)),
            pl.BlockSpec((d_in, d_out), lambda i: (0, 0)),
        ],
        out_specs=pl.BlockSpec((blk, d_out), lambda i: (i, 0)),
        out_shape=jax.ShapeDtypeStruct((n, d_out), jnp.float32),
    )(x, w)


def _tc_b2(h, dsum):
    n, d = h.shape
    blk = 2000

    def body(h_ref, d_ref, out_ref):
        dinv = 1.0 / jnp.sqrt(d_ref[...] + 1.0)
        out_ref[...] = h_ref[...] * dinv

    return pl.pallas_call(
        body,
        grid=(n // blk,),
        in_specs=[
            pl.BlockSpec((blk, d), lambda i: (i, 0)),
            pl.BlockSpec((blk, 1), lambda i: (i, 0)),
        ],
        out_specs=pl.BlockSpec((blk, d), lambda i: (i, 0)),
        out_shape=jax.ShapeDtypeStruct((n, d), jnp.float32),
    )(h, dsum)


def _tc_d(acc, g, dsum, b2):
    n, d = g.shape
    blk = 2000

    def body(acc_ref, g_ref, d_ref, b_ref, out_ref):
        dinv = 1.0 / jnp.sqrt(d_ref[...] + 1.0)
        tot = acc_ref[0] + acc_ref[1] - g_ref[...]
        out_ref[...] = tot * dinv + b_ref[...]

    return pl.pallas_call(
        body,
        grid=(n // blk,),
        in_specs=[
            pl.BlockSpec((NC, blk, d), lambda i: (0, i, 0)),
            pl.BlockSpec((blk, d), lambda i: (i, 0)),
            pl.BlockSpec((blk, 1), lambda i: (i, 0)),
            pl.BlockSpec((1, d), lambda i: (0, 0)),
        ],
        out_specs=pl.BlockSpec((blk, d), lambda i: (i, 0)),
        out_shape=jax.ShapeDtypeStruct((n, d), jnp.float32),
    )(acc, g, dsum, b2)


def kernel(x, edge_index, W, b):
    n, _ = x.shape
    e = edge_index.shape[1]
    assert e % EB == 0 and n % LANES == 0
    h = _tc_b1(x, W)
    deg_p = _deg_pass(edge_index, n)
    dsum = (deg_p[0, 0] + deg_p[1, 0]).reshape(n, 1)
    g = _tc_b2(h, dsum)
    acc = _gs_pass(g, edge_index)
    return _tc_d(acc, g, dsum, b.reshape(1, -1))
